# baseline (speedup 1.0000x reference)
import jax
import jax.numpy as jnp
from jax import lax
from jax.experimental import pallas as pl
from jax.experimental.pallas import tpu as pltpu
from jax.experimental.pallas import tpu_sc as plsc

NC = 2
NS = 16
NW = NC * NS
L = 16

ROWS = 320
WORDS = 384
WIN = 1280


def _build_kernel(adj0, adj1, a_out, sbig, dbig, abuf, sem0, sem1):
    E = adj0.shape[0]
    n_win = E // WIN
    cpw = WIN // L
    wid = lax.axis_index("s") * NC + lax.axis_index("c")
    lo = wid * ROWS
    hi = lo + ROWS

    def _z(i, _):
        for u in range(8):
            abuf[pl.ds((i * 8 + u) * L, L)] = jnp.zeros((L,), jnp.int32)
        return ()

    lax.fori_loop(0, ROWS * WORDS // (8 * L), _z, ())

    def _issue(g, off, sem):
        pltpu.async_copy(adj0.at[pl.ds(g * WIN, WIN)],
                         sbig.at[pl.ds(off, WIN)], sem)
        pltpu.async_copy(adj1.at[pl.ds(g * WIN, WIN)],
                         dbig.at[pl.ds(off, WIN)], sem)

    def _wait(off, sem):
        pltpu.make_async_copy(adj0.at[pl.ds(0, WIN)],
                              sbig.at[pl.ds(off, WIN)], sem).wait()
        pltpu.make_async_copy(adj1.at[pl.ds(0, WIN)],
                              dbig.at[pl.ds(off, WIN)], sem).wait()

    _issue(0, 0, sem0)

    def _chunk(c, _):
        qpw = cpw // 16
        g = c // qpw
        phase = c % (2 * qpw)

        @pl.when(phase == 0)
        def _():
            _wait(0, sem0)

            @pl.when(g + 1 < n_win)
            def _():
                _issue(g + 1, WIN, sem1)

        @pl.when(phase == qpw)
        def _():
            _wait(WIN, sem1)

            @pl.when(g + 1 < n_win)
            def _():
                _issue(g + 1, 0, sem0)

        off = phase * 16 * L
        ss = [sbig[pl.ds(off + u * L, L)] for u in range(16)]
        ms = [jnp.logical_and(sv >= lo, sv < hi) for sv in ss]
        ns = [plsc.all_reduce_population_count(mv)[0] for mv in ms]

        def _hits(o2, n_hit, m):
            @pl.when(n_hit > 0)
            def _():
                def _hit(i, mv):
                    l = plsc.all_reduce_ffs(mv)[0]
                    se = sbig[pl.ds(o2 + l, L)][0]
                    de = dbig[pl.ds(o2 + l, L)][0]
                    w = de >> 5
                    wa = (w >> 4) * L
                    lane = w - wa
                    fa = (se - lo) * WORDS + wa
                    bit = jnp.int32(1) << (de & 31)
                    vec = abuf[pl.ds(fa, L)]
                    abuf[pl.ds(fa, L)] = jnp.where(
                        lax.iota(jnp.int32, L) == lane, vec | bit, vec)
                    return mv & (lax.iota(jnp.int32, L) != l)

                lax.fori_loop(0, n_hit, _hit, m)

        for u in range(16):
            _hits(off + u * L, ns[u], ms[u])
        return ()

    lax.fori_loop(0, E // (16 * L), _chunk, ())

    pltpu.sync_copy(abuf, a_out.at[pl.ds(lo * WORDS, ROWS * WORDS)])


def _intersect_kernel(a_hbm, t0, t1, x_hbm, xf_hbm, xcn, xi, xj,
                      idx0, idx1, rows_i, rows_j, fbuf, acc, w16, anyb,
                      xrow, sem):
    D = x_hbm.shape[1]
    BH = idx0.shape[0]
    wid = lax.axis_index("s") * NC + lax.axis_index("c")

    for h in (0, 1):
        base = wid * 2 * BH + h * BH

        pltpu.sync_copy(t0.at[pl.ds(base, BH)], idx0)
        pltpu.sync_copy(t1.at[pl.ds(base, BH)], idx1)

        pltpu.async_copy(x_hbm.at[idx0], fbuf, sem).wait()
        pltpu.sync_copy(fbuf, xi.at[pl.ds(base, BH)])
        pltpu.async_copy(x_hbm.at[idx1], fbuf, sem).wait()
        pltpu.sync_copy(fbuf, xj.at[pl.ds(base, BH)])

        pltpu.async_copy(a_hbm.at[idx0], rows_i, sem).wait()
        pltpu.async_copy(a_hbm.at[idx1], rows_j, sem).wait()

        def _zr(r, _):
            def _zc(c, _):
                acc[r, pl.ds(c * L, L)] = jnp.zeros((L,), jnp.float32)
                return ()
            return lax.fori_loop(0, D // L, _zc, ())

        lax.fori_loop(0, BH, _zr, ())

        def _tgt(b, _):
            def _chunk(c, anyv):
                wi = rows_i[b, pl.ds(c * L, L)] & rows_j[b, pl.ds(c * L, L)]
                w16[pl.ds(c * L, L)] = wi
                return anyv | wi

            anyv = lax.fori_loop(0, WORDS // L, _chunk,
                                 jnp.zeros((L,), jnp.int32))
            anyb[pl.ds(0, L)] = anyv

            def _lane(lam, _):
                lane_or = anyb[pl.ds(lam, L)][0]

                @pl.when(lane_or != 0)
                def _():
                    def _wt(t, _):
                        w = t * L + lam
                        word = w16[pl.ds(w, L)][0]

                        @pl.when(word != 0)
                        def _():
                            def _bit(j, _):
                                @pl.when(((word >> j) & 1) != 0)
                                def _():
                                    k = (w << 5) | j
                                    pltpu.sync_copy(
                                        xf_hbm.at[pl.ds(k * D, D)], xrow)

                                    def _addc(cc, _):
                                        sl = pl.ds(cc * L, L)
                                        acc[b, sl] = acc[b, sl] + xrow[sl]
                                        return ()

                                    lax.fori_loop(0, D // L, _addc, ())
                                return ()

                            lax.fori_loop(0, 32, _bit, ())
                        return ()

                    lax.fori_loop(0, WORDS // L, _wt, ())
                return ()

            return lax.fori_loop(0, L, _lane, ())

        lax.fori_loop(0, BH, _tgt, ())

        pltpu.sync_copy(acc, xcn.at[pl.ds(base, BH)])


def _mlp_kernel(xcn_ref, xi_ref, xj_ref, beta_ref,
                w1_ref, b1_ref, w2_ref, b2_ref, w3_ref, b3_ref,
                u1_ref, c1_ref, u2_ref, c2_ref,
                l1_ref, d1_ref, l2_ref, d2_ref, o_ref):
    xij = jnp.maximum((xi_ref[...] * xj_ref[...]) @ u1_ref[...] + c1_ref[...],
                      0.0)
    xij = xij @ u2_ref[...] + c2_ref[...]
    h = jnp.maximum(xcn_ref[...] @ w1_ref[...] + b1_ref[...], 0.0)
    h = jnp.maximum(h @ w2_ref[...] + b2_ref[...], 0.0)
    h = h @ w3_ref[...] + b3_ref[...]
    z = jnp.maximum((h * beta_ref[...] + xij) @ l1_ref[...] + d1_ref[...], 0.0)
    o_ref[...] = (z @ l2_ref[...] + d2_ref[...]).astype(jnp.float32)


def kernel(x, adj, tar_ei, beta,
           xcn_W1, xcn_b1, xcn_W2, xcn_b2, xcn_W3, xcn_b3,
           xij_W1, xij_b1, xij_W2, xij_b2,
           lin_W1, lin_b1, lin_W2, lin_b2):
    N, D = x.shape
    B = tar_ei.shape[1]
    BT = B // NW

    adj0 = adj[0]
    adj1 = adj[1]
    t0 = tar_ei[0]
    t1 = tar_ei[1]
    x_flat = x.reshape(N * D)

    mesh = plsc.VectorSubcoreMesh(core_axis_name="c", subcore_axis_name="s")

    build = pl.kernel(
        _build_kernel,
        mesh=mesh,
        compiler_params=pltpu.CompilerParams(needs_layout_passes=False),
        out_type=jax.ShapeDtypeStruct((NW * ROWS * WORDS,), jnp.int32),
        scratch_types=[
            pltpu.VMEM((2 * WIN + L,), jnp.int32),
            pltpu.VMEM((2 * WIN + L,), jnp.int32),
            pltpu.VMEM((ROWS * WORDS,), jnp.int32),
            pltpu.SemaphoreType.DMA,
            pltpu.SemaphoreType.DMA,
        ],
    )
    a_bits = build(adj0, adj1).reshape(NW * ROWS, WORDS)

    intersect = pl.kernel(
        _intersect_kernel,
        mesh=mesh,
        compiler_params=pltpu.CompilerParams(needs_layout_passes=False),
        out_type=(
            jax.ShapeDtypeStruct((B, D), jnp.float32),
            jax.ShapeDtypeStruct((B, D), jnp.float32),
            jax.ShapeDtypeStruct((B, D), jnp.float32),
        ),
        scratch_types=[
            pltpu.VMEM((BT // 2,), jnp.int32),
            pltpu.VMEM((BT // 2,), jnp.int32),
            pltpu.VMEM((BT // 2, WORDS), jnp.int32),
            pltpu.VMEM((BT // 2, WORDS), jnp.int32),
            pltpu.VMEM((BT // 2, D), jnp.float32),
            pltpu.VMEM((BT // 2, D), jnp.float32),
            pltpu.VMEM((WORDS + L,), jnp.int32),
            pltpu.VMEM((2 * L,), jnp.int32),
            pltpu.VMEM((D,), jnp.float32),
            pltpu.SemaphoreType.DMA,
        ],
    )
    xcn, xi, xj = intersect(a_bits, t0, t1, x, x_flat)

    out = pl.pallas_call(
        _mlp_kernel,
        out_shape=jax.ShapeDtypeStruct((B, lin_W2.shape[1]), jnp.float32),
    )(xcn, xi, xj, beta.reshape(1, 1),
      xcn_W1, xcn_b1, xcn_W2, xcn_b2, xcn_W3, xcn_b3,
      xij_W1, xij_b1, xij_W2, xij_b2,
      lin_W1, lin_b1, lin_W2, lin_b2)
    return out


# speedup vs baseline: 1.1605x; 1.1605x over previous
import jax
import jax.numpy as jnp
from jax import lax
from jax.experimental import pallas as pl
from jax.experimental.pallas import tpu as pltpu
from jax.experimental.pallas import tpu_sc as plsc

NC = 2
NS = 16
NW = NC * NS
L = 16

ROWS = 320
WORDS = 384
WIN = 1280


def _build_kernel(adj0, adj1, a_out, sbig, dbig, abuf, sem0, sem1):
    E = adj0.shape[0]
    n_win = E // WIN
    cpw = WIN // L
    wid = lax.axis_index("s") * NC + lax.axis_index("c")
    lo = wid * ROWS
    hi = lo + ROWS

    def _z(i, _):
        for u in range(8):
            abuf[pl.ds((i * 8 + u) * L, L)] = jnp.zeros((L,), jnp.int32)
        return ()

    lax.fori_loop(0, ROWS * WORDS // (8 * L), _z, ())

    def _issue(g, off, sem):
        pltpu.async_copy(adj0.at[pl.ds(g * WIN, WIN)],
                         sbig.at[pl.ds(off, WIN)], sem)
        pltpu.async_copy(adj1.at[pl.ds(g * WIN, WIN)],
                         dbig.at[pl.ds(off, WIN)], sem)

    def _wait(off, sem):
        pltpu.make_async_copy(adj0.at[pl.ds(0, WIN)],
                              sbig.at[pl.ds(off, WIN)], sem).wait()
        pltpu.make_async_copy(adj1.at[pl.ds(0, WIN)],
                              dbig.at[pl.ds(off, WIN)], sem).wait()

    _issue(0, 0, sem0)

    def _chunk(c, _):
        qpw = cpw // 16
        g = c // qpw
        phase = c % (2 * qpw)

        @pl.when(phase == 0)
        def _():
            _wait(0, sem0)

            @pl.when(g + 1 < n_win)
            def _():
                _issue(g + 1, WIN, sem1)

        @pl.when(phase == qpw)
        def _():
            _wait(WIN, sem1)

            @pl.when(g + 1 < n_win)
            def _():
                _issue(g + 1, 0, sem0)

        off = phase * 16 * L
        ss = [sbig[pl.ds(off + u * L, L)] for u in range(16)]
        ms = [jnp.logical_and(sv >= lo, sv < hi) for sv in ss]
        ns = [plsc.all_reduce_population_count(mv)[0] for mv in ms]

        def _hits(o2, n_hit, m, sv):
            @pl.when(n_hit == 1)
            def _():
                dd = dbig[pl.ds(o2, L)]
                aw = (sv - lo) * WORDS + (dd >> 5)
                bitv = jnp.int32(1) << (dd & 31)
                old = plsc.load_gather(abuf.at[...], [aw], mask=m)
                plsc.store_scatter(abuf.at[...], [aw], old | bitv, mask=m)

            @pl.when(n_hit > 1)
            def _():
                def _hit(i, mv):
                    l = plsc.all_reduce_ffs(mv)[0]
                    se = sbig[pl.ds(o2 + l, L)][0]
                    de = dbig[pl.ds(o2 + l, L)][0]
                    w = de >> 5
                    wa = (w >> 4) * L
                    lane = w - wa
                    fa = (se - lo) * WORDS + wa
                    bit = jnp.int32(1) << (de & 31)
                    vec = abuf[pl.ds(fa, L)]
                    abuf[pl.ds(fa, L)] = jnp.where(
                        lax.iota(jnp.int32, L) == lane, vec | bit, vec)
                    return mv & (lax.iota(jnp.int32, L) != l)

                lax.fori_loop(0, n_hit, _hit, m)

        for u in range(16):
            _hits(off + u * L, ns[u], ms[u], ss[u])
        return ()

    lax.fori_loop(0, E // (16 * L), _chunk, ())

    pltpu.sync_copy(abuf, a_out.at[pl.ds(lo * WORDS, ROWS * WORDS)])


def _intersect_kernel(a_hbm, t0, t1, x_hbm, xf_hbm, xcn, xi, xj,
                      idx0, idx1, rows_i, rows_j, fbuf, acc, w16, anyb,
                      xrow, sem):
    D = x_hbm.shape[1]
    BH = idx0.shape[0]
    wid = lax.axis_index("s") * NC + lax.axis_index("c")

    for h in (0, 1):
        base = wid * 2 * BH + h * BH

        pltpu.sync_copy(t0.at[pl.ds(base, BH)], idx0)
        pltpu.sync_copy(t1.at[pl.ds(base, BH)], idx1)

        pltpu.async_copy(x_hbm.at[idx0], fbuf, sem).wait()
        pltpu.sync_copy(fbuf, xi.at[pl.ds(base, BH)])
        pltpu.async_copy(x_hbm.at[idx1], fbuf, sem).wait()
        pltpu.sync_copy(fbuf, xj.at[pl.ds(base, BH)])

        pltpu.async_copy(a_hbm.at[idx0], rows_i, sem).wait()
        pltpu.async_copy(a_hbm.at[idx1], rows_j, sem).wait()

        def _zr(r, _):
            def _zc(c, _):
                acc[r, pl.ds(c * L, L)] = jnp.zeros((L,), jnp.float32)
                return ()
            return lax.fori_loop(0, D // L, _zc, ())

        lax.fori_loop(0, BH, _zr, ())

        def _tgt(b, _):
            def _chunk(c, anyv):
                wi = rows_i[b, pl.ds(c * L, L)] & rows_j[b, pl.ds(c * L, L)]
                w16[pl.ds(c * L, L)] = wi
                return anyv | wi

            anyv = lax.fori_loop(0, WORDS // L, _chunk,
                                 jnp.zeros((L,), jnp.int32))
            anyb[pl.ds(0, L)] = anyv

            def _lane(lam, _):
                lane_or = anyb[pl.ds(lam, L)][0]

                @pl.when(lane_or != 0)
                def _():
                    def _wt(t, _):
                        w = t * L + lam
                        word = w16[pl.ds(w, L)][0]

                        @pl.when(word != 0)
                        def _():
                            def _bit(j, _):
                                @pl.when(((word >> j) & 1) != 0)
                                def _():
                                    k = (w << 5) | j
                                    pltpu.sync_copy(
                                        xf_hbm.at[pl.ds(k * D, D)], xrow)

                                    def _addc(cc, _):
                                        sl = pl.ds(cc * L, L)
                                        acc[b, sl] = acc[b, sl] + xrow[sl]
                                        return ()

                                    lax.fori_loop(0, D // L, _addc, ())
                                return ()

                            lax.fori_loop(0, 32, _bit, ())
                        return ()

                    lax.fori_loop(0, WORDS // L, _wt, ())
                return ()

            return lax.fori_loop(0, L, _lane, ())

        lax.fori_loop(0, BH, _tgt, ())

        pltpu.sync_copy(acc, xcn.at[pl.ds(base, BH)])


def _mlp_kernel(xcn_ref, xi_ref, xj_ref, beta_ref,
                w1_ref, b1_ref, w2_ref, b2_ref, w3_ref, b3_ref,
                u1_ref, c1_ref, u2_ref, c2_ref,
                l1_ref, d1_ref, l2_ref, d2_ref, o_ref):
    xij = jnp.maximum((xi_ref[...] * xj_ref[...]) @ u1_ref[...] + c1_ref[...],
                      0.0)
    xij = xij @ u2_ref[...] + c2_ref[...]
    h = jnp.maximum(xcn_ref[...] @ w1_ref[...] + b1_ref[...], 0.0)
    h = jnp.maximum(h @ w2_ref[...] + b2_ref[...], 0.0)
    h = h @ w3_ref[...] + b3_ref[...]
    z = jnp.maximum((h * beta_ref[...] + xij) @ l1_ref[...] + d1_ref[...], 0.0)
    o_ref[...] = (z @ l2_ref[...] + d2_ref[...]).astype(jnp.float32)


def kernel(x, adj, tar_ei, beta,
           xcn_W1, xcn_b1, xcn_W2, xcn_b2, xcn_W3, xcn_b3,
           xij_W1, xij_b1, xij_W2, xij_b2,
           lin_W1, lin_b1, lin_W2, lin_b2):
    N, D = x.shape
    B = tar_ei.shape[1]
    BT = B // NW

    adj0 = adj[0]
    adj1 = adj[1]
    t0 = tar_ei[0]
    t1 = tar_ei[1]
    x_flat = x.reshape(N * D)

    mesh = plsc.VectorSubcoreMesh(core_axis_name="c", subcore_axis_name="s")

    build = pl.kernel(
        _build_kernel,
        mesh=mesh,
        compiler_params=pltpu.CompilerParams(needs_layout_passes=False),
        out_type=jax.ShapeDtypeStruct((NW * ROWS * WORDS,), jnp.int32),
        scratch_types=[
            pltpu.VMEM((2 * WIN + L,), jnp.int32),
            pltpu.VMEM((2 * WIN + L,), jnp.int32),
            pltpu.VMEM((ROWS * WORDS,), jnp.int32),
            pltpu.SemaphoreType.DMA,
            pltpu.SemaphoreType.DMA,
        ],
    )
    a_bits = build(adj0, adj1).reshape(NW * ROWS, WORDS)

    intersect = pl.kernel(
        _intersect_kernel,
        mesh=mesh,
        compiler_params=pltpu.CompilerParams(needs_layout_passes=False),
        out_type=(
            jax.ShapeDtypeStruct((B, D), jnp.float32),
            jax.ShapeDtypeStruct((B, D), jnp.float32),
            jax.ShapeDtypeStruct((B, D), jnp.float32),
        ),
        scratch_types=[
            pltpu.VMEM((BT // 2,), jnp.int32),
            pltpu.VMEM((BT // 2,), jnp.int32),
            pltpu.VMEM((BT // 2, WORDS), jnp.int32),
            pltpu.VMEM((BT // 2, WORDS), jnp.int32),
            pltpu.VMEM((BT // 2, D), jnp.float32),
            pltpu.VMEM((BT // 2, D), jnp.float32),
            pltpu.VMEM((WORDS + L,), jnp.int32),
            pltpu.VMEM((2 * L,), jnp.int32),
            pltpu.VMEM((D,), jnp.float32),
            pltpu.SemaphoreType.DMA,
        ],
    )
    xcn, xi, xj = intersect(a_bits, t0, t1, x, x_flat)

    out = pl.pallas_call(
        _mlp_kernel,
        out_shape=jax.ShapeDtypeStruct((B, lin_W2.shape[1]), jnp.float32),
    )(xcn, xi, xj, beta.reshape(1, 1),
      xcn_W1, xcn_b1, xcn_W2, xcn_b2, xcn_W3, xcn_b3,
      xij_W1, xij_b1, xij_W2, xij_b2,
      lin_W1, lin_b1, lin_W2, lin_b2)
    return out
